# baseline (speedup 1.0000x reference)
import jax
import jax.numpy as jnp
from jax import lax
from jax.experimental import pallas as pl
from jax.experimental.pallas import tpu as pltpu
from jax.experimental.pallas import tpu_sc as plsc

_THRESH = 0.05
_TOP_N = 1000
_NT = 16
_SHARD = 1310720 // _NT
_NV = _SHARD // 16
_OUT_PAD = 1024
_MIN32 = -(2 ** 31)


def _sigmoid_mask_t(x):
    N, C, HW = x.shape

    def body(x_ref, o_ref):
        s = jax.nn.sigmoid(jnp.transpose(x_ref[0], (1, 0)))
        o_ref[0] = jnp.where(s > _THRESH, s, -1.0)

    blk = 2048
    return pl.pallas_call(
        body,
        grid=(N, HW // blk),
        in_specs=[pl.BlockSpec((1, C, blk), lambda i, j: (i, 0, j))],
        out_specs=pl.BlockSpec((1, blk, C), lambda i, j: (i, j, 0)),
        out_shape=jax.ShapeDtypeStruct((N, HW, C), x.dtype),
    )(x)


def _sc_body(masked_hbm, outS_hbm, outI_hbm,
             shard, hist, local, tot, staging, bufK, bufI, othersK, othersI,
             pos_pub, rank_buf, score_buf, idx_buf, cntcopy,
             histgrid_sm, candK_sm, candI_sm, cntgrid_sm,
             outS_sm, outI_sm):
    c = lax.axis_index("c")
    s = lax.axis_index("s")
    lanes = lax.iota(jnp.int32, 16)
    lane_str = lanes * 2049
    ones = jnp.ones((16,), jnp.int32)

    def splat(v):
        return jnp.full((16,), v, jnp.int32)

    def extract(vec, lane):
        return jnp.max(jnp.where(lanes == lane, vec, _MIN32))

    pltpu.sync_copy(masked_hbm.at[c, pl.ds(s * _SHARD, _SHARD)], shard)

    def clear_hist():
        def clr(w, _):
            for k in range(8):
                hist[pl.ds(w * 128 + k * 16, 16)] = jnp.zeros((16,), jnp.int32)
            return 0

        lax.fori_loop(0, 258, clr, 0)

    def reduce_and_decide(R, want_take):
        def red_w(w, _):
            acc = hist[pl.ds(w * 16, 16)]
            for lq in range(1, 16):
                acc = acc + hist[pl.ds(lq * 2049 + w * 16, 16)]
            local[pl.ds(w * 16, 16)] = acc
            return 0

        lax.fori_loop(0, 128, red_w, 0)

        pltpu.sync_copy(local, histgrid_sm.at[pl.ds(s * 2048, 2048)])
        plsc.subcore_barrier()

        def zt(w, _):
            tot[pl.ds(w * 16, 16)] = jnp.zeros((16,), jnp.int32)
            return 0

        lax.fori_loop(0, 128, zt, 0)

        def acc_r(r, _):
            pltpu.sync_copy(histgrid_sm.at[pl.ds(r * 2048, 2048)], staging)

            def aw(w, _):
                tot[pl.ds(w * 16, 16)] = (tot[pl.ds(w * 16, 16)]
                                          + staging[pl.ds(w * 16, 16)])
                return 0

            lax.fori_loop(0, 128, aw, 0)
            return 0

        lax.fori_loop(0, 16, acc_r, 0)

        def walk(i, wc):
            R_w, b_w, done = wc
            blk = 127 - i
            v = tot[pl.ds(blk * 16, 16)]
            ssum = jnp.sum(v)
            csum = plsc.cumsum(v)
            sfx = splat(ssum) - csum + v
            ge = sfx >= splat(R_w)
            m = jnp.max(plsc.all_reduce_population_count(ge)) - 1
            found = jnp.logical_and(jnp.logical_not(done), R_w <= ssum)
            vm = extract(v, m)
            sfxm = extract(sfx, m)
            R_in = R_w - (sfxm - vm)
            b_new = jnp.where(found, blk * 16 + m, b_w)
            R_new = jnp.where(found, R_in,
                              jnp.where(done, R_w, R_w - ssum))
            return (R_new, b_new, jnp.logical_or(done, found))

        R_f, b_l, _ = lax.fori_loop(
            0, 128, walk, (R, jnp.int32(0), jnp.bool_(False)))

        take = jnp.int32(0)
        if want_take:
            b3a = (b_l // 16) * 16
            b3m = b_l - b3a

            def ev_r(r, ev):
                pltpu.sync_copy(histgrid_sm.at[pl.ds(r * 2048 + b3a, 16)],
                                staging.at[pl.ds(0, 16)])
                val = extract(staging[pl.ds(0, 16)], b3m)
                return jnp.where(lanes == r, splat(val), ev)

            evec = lax.fori_loop(0, 16, ev_r, jnp.zeros((16,), jnp.int32))
            pecx = plsc.cumsum(evec) - evec
            tk = jnp.clip(splat(R_f) - pecx, 0, evec)
            take = extract(tk, s)
        plsc.subcore_barrier()
        return R_f, b_l, take

    clear_hist()

    def scan0(i, _):
        for k in range(8):
            off = i * 128 + k * 16
            b = lax.bitcast_convert_type(shard[pl.ds(off, 16)], jnp.int32)
            key = jnp.where(b < 0,
                            jnp.bitwise_xor(jnp.bitwise_not(b), _MIN32), b)
            shard[pl.ds(off, 16)] = lax.bitcast_convert_type(key, jnp.float32)
            u = jnp.bitwise_xor(key, _MIN32)
            digit = lax.shift_right_logical(u, 21)
            plsc.addupdate_scatter(hist, [lane_str + digit], ones)
        return 0

    lax.fori_loop(0, _NV // 8, scan0, 0)
    R_run, b_run, _ = reduce_and_decide(jnp.int32(_TOP_N), False)
    pfx_u = lax.shift_left(b_run, 21)
    my_take = jnp.int32(0)

    for sh, mask in ((10, -2097152), (0, -1024)):
        clear_hist()
        pfx_s = splat(pfx_u)
        mb_s = splat(mask)

        def scan(i, _, _sh=sh, _pfx_s=pfx_s, _mb_s=mb_s):
            ks = []
            matches = []
            anym = None
            for k in range(8):
                kk = lax.bitcast_convert_type(
                    shard[pl.ds(i * 128 + k * 16, 16)], jnp.int32)
                u = jnp.bitwise_xor(kk, _MIN32)
                m_ = jnp.bitwise_and(u, _mb_s) == _pfx_s
                ks.append(u)
                matches.append(m_)
                anym = m_ if anym is None else jnp.logical_or(anym, m_)

            @pl.when(jnp.any(anym))
            def _do():
                for k in range(8):
                    d = ks[k] if _sh == 0 else lax.shift_right_logical(
                        ks[k], _sh)
                    digit = jnp.bitwise_and(d, 2047)
                    plsc.addupdate_scatter(hist, [lane_str + digit], ones,
                                           mask=matches[k])
            return 0

        lax.fori_loop(0, _NV // 8, scan, 0)
        R_run, b_run, take_l = reduce_and_decide(R_run, sh == 0)
        pfx_u = jnp.bitwise_or(pfx_u, lax.shift_left(b_run, sh))
        if sh == 0:
            my_take = take_l
    R3 = R_run
    t_key = jnp.bitwise_xor(pfx_u, _MIN32)

    t_s = splat(t_key)
    take_s = splat(my_take)
    base = s * _SHARD

    def fscan(i, carry):
        cnt_s, cntE_s = carry
        ks = []
        gts = []
        eqs = []
        anyv = None
        for k in range(8):
            kk = lax.bitcast_convert_type(
                shard[pl.ds(i * 128 + k * 16, 16)], jnp.int32)
            g = kk > t_s
            e = kk == t_s
            ks.append(kk)
            gts.append(g)
            eqs.append(e)
            h = jnp.logical_or(g, e)
            anyv = h if anyv is None else jnp.logical_or(anyv, h)

        def slow(cc):
            cnt_s, cntE_s = cc
            for k in range(8):
                gt, eq = gts[k], eqs[k]
                idxv = splat(base + i * 128 + k * 16) + lanes
                gti = gt.astype(jnp.int32)
                pg = plsc.cumsum(gti) - gti
                plsc.store_scatter(bufK, [cnt_s + pg], ks[k], mask=gt)
                plsc.store_scatter(bufI, [cnt_s + pg], idxv, mask=gt)
                ng = plsc.all_reduce_population_count(gt)
                eqi = eq.astype(jnp.int32)
                pe = plsc.cumsum(eqi) - eqi
                eff = jnp.logical_and(eq, (cntE_s + pe) < take_s)
                effi = eff.astype(jnp.int32)
                pf = plsc.cumsum(effi) - effi
                off2 = cnt_s + ng + pf
                plsc.store_scatter(bufK, [off2], ks[k], mask=eff)
                plsc.store_scatter(bufI, [off2], idxv, mask=eff)
                ne = plsc.all_reduce_population_count(eff)
                cnt_s = cnt_s + ng + ne
                cntE_s = cntE_s + ne
            return (cnt_s, cntE_s)

        return lax.cond(jnp.any(anyv), slow, lambda cc: cc, (cnt_s, cntE_s))

    cnt_spl, _ = lax.fori_loop(
        0, _NV // 8, fscan,
        (jnp.zeros((16,), jnp.int32), jnp.zeros((16,), jnp.int32)))
    cnt = jnp.max(cnt_spl)

    local[pl.ds(0, 16)] = splat(cnt)
    pltpu.sync_copy(local.at[pl.ds(0, 16)], cntgrid_sm.at[pl.ds(s * 16, 16)])
    plsc.subcore_barrier()
    pltpu.sync_copy(cntgrid_sm, cntcopy)

    def cv_r(r, cv):
        row = cntcopy[pl.ds(r * 16, 16)]
        return jnp.where(lanes == r, row, cv)

    cvec = lax.fori_loop(0, 16, cv_r, jnp.zeros((16,), jnp.int32))
    offs = plsc.cumsum(cvec) - cvec
    my_off = extract(offs, s)

    for j in range(8):
        for w in range(8):
            kk = j * 128 + w * 16
            kvec = splat(kk) + lanes
            in_real = kvec < splat(cnt)
            pos = jnp.where(in_real, splat(my_off) + kvec,
                            splat(_OUT_PAD) + kvec)
            pos_pub[j, pl.ds(w * 16, 16)] = pos
    for j in range(8):
        pltpu.sync_copy(bufK.at[pl.ds(j * 128, 128)], candK_sm.at[pos_pub.at[j]])
        pltpu.sync_copy(bufI.at[pl.ds(j * 128, 128)], candI_sm.at[pos_pub.at[j]])
    plsc.subcore_barrier()
    pltpu.sync_copy(candK_sm.at[pl.ds(0, _OUT_PAD)], othersK)
    pltpu.sync_copy(candI_sm.at[pl.ds(0, _OUT_PAD)], othersI)

    my_lo = s * 64

    def rank_j(j, _):
        p = my_lo + j
        pa = (p // 16) * 16
        pm = p - pa
        kj = extract(othersK[pl.ds(pa, 16)], pm)
        ij = extract(othersI[pl.ds(pa, 16)], pm)
        kj_s = splat(kj)
        ij_s = splat(ij)

        def sweep(w, acc):
            ok = othersK[pl.ds(w * 16, 16)]
            oi = othersI[pl.ds(w * 16, 16)]
            valid = (splat(w * 16) + lanes) < _TOP_N
            beat = jnp.logical_or(
                ok > kj_s, jnp.logical_and(ok == kj_s, oi < ij_s))
            return acc + plsc.all_reduce_population_count(
                jnp.logical_and(beat, valid))

        racc = lax.fori_loop(0, 63, sweep, jnp.zeros((16,), jnp.int32))
        rank = jnp.max(racc)
        rank = jnp.where(p < _TOP_N, rank, p)
        plsc.store_scatter(rank_buf, [splat(j)], splat(rank),
                           mask=lanes == 0)
        return 0

    lax.fori_loop(0, 64, rank_j, 0)

    for m in range(4):
        kv = lax.bitcast_convert_type(othersK[pl.ds(my_lo + m * 16, 16)], jnp.int32)
        bits = jnp.where(kv < 0,
                         jnp.bitwise_not(jnp.bitwise_xor(kv, _MIN32)), kv)
        score_buf[pl.ds(m * 16, 16)] = lax.bitcast_convert_type(bits, jnp.float32)
        idx_buf[pl.ds(m * 16, 16)] = othersI[pl.ds(my_lo + m * 16, 16)]
    pltpu.sync_copy(score_buf, outS_sm.at[rank_buf])
    pltpu.sync_copy(idx_buf, outI_sm.at[rank_buf])
    plsc.subcore_barrier()

    @pl.when(s == 0)
    def _write_out():
        pltpu.sync_copy(outS_sm, outS_hbm.at[c])
        pltpu.sync_copy(outI_sm, outI_hbm.at[c])


def _sc_topk(masked):
    mesh = plsc.VectorSubcoreMesh(core_axis_name="c", subcore_axis_name="s")
    f32 = jnp.float32
    i32 = jnp.int32
    return pl.kernel(
        _sc_body,
        out_type=[jax.ShapeDtypeStruct((2, _OUT_PAD), f32),
                  jax.ShapeDtypeStruct((2, _OUT_PAD), i32)],
        mesh=mesh,
        compiler_params=pltpu.CompilerParams(needs_layout_passes=False),
        scratch_types=[
            pltpu.VMEM((_SHARD,), f32),
            pltpu.VMEM((33024,), i32),
            pltpu.VMEM((2048,), i32),
            pltpu.VMEM((2048,), i32),
            pltpu.VMEM((2048,), i32),
            pltpu.VMEM((1024,), i32),
            pltpu.VMEM((1024,), i32),
            pltpu.VMEM((1024,), i32),
            pltpu.VMEM((1024,), i32),
            pltpu.VMEM((8, 128), i32),
            pltpu.VMEM((64,), i32),
            pltpu.VMEM((64,), f32),
            pltpu.VMEM((64,), i32),
            pltpu.VMEM((256,), i32),
            pltpu.VMEM_SHARED((32768,), i32),
            pltpu.VMEM_SHARED((2 * _OUT_PAD,), i32),
            pltpu.VMEM_SHARED((2 * _OUT_PAD,), i32),
            pltpu.VMEM_SHARED((256,), i32),
            pltpu.VMEM_SHARED((_OUT_PAD,), f32),
            pltpu.VMEM_SHARED((_OUT_PAD,), i32),
        ],
    )(masked)


def kernel(locations, box_cls_set, box_regression, centerness, image_sizes):
    N, C, H, W = box_cls_set.shape
    HW = H * W
    masked = _sigmoid_mask_t(
        box_cls_set.reshape(N, C, HW)).reshape(N, HW * C)

    outS, outI = _sc_topk(masked)
    top_scores = outS[:, :_TOP_N]
    top_idx = outI[:, :_TOP_N]

    box_reg = jnp.transpose(box_regression, (0, 2, 3, 1)).reshape(N, HW, 4)
    loc_idx = top_idx // C
    labels = top_idx % C + 1
    valid = top_scores > _THRESH
    per_loc = locations[loc_idx]
    per_reg = jnp.take_along_axis(box_reg, loc_idx[..., None], axis=1)
    x1 = per_loc[..., 0] - per_reg[..., 0]
    y1 = per_loc[..., 1] - per_reg[..., 1]
    x2 = per_loc[..., 0] + per_reg[..., 2]
    y2 = per_loc[..., 1] + per_reg[..., 3]
    h_img = image_sizes[:, 0].astype(jnp.float32)[:, None]
    w_img = image_sizes[:, 1].astype(jnp.float32)[:, None]
    x1 = jnp.clip(x1, 0.0, w_img - 1.0)
    x2 = jnp.clip(x2, 0.0, w_img - 1.0)
    y1 = jnp.clip(y1, 0.0, h_img - 1.0)
    y2 = jnp.clip(y2, 0.0, h_img - 1.0)
    detections = jnp.stack([x1, y1, x2, y2], axis=-1)
    ws = x2 - x1 + 1.0
    hs = y2 - y1 + 1.0
    keep = (ws >= 0) & (hs >= 0)
    scores = jnp.where(valid & keep, top_scores, 0.0)
    return detections, scores, labels, per_loc


# speedup vs baseline: 1.1263x; 1.1263x over previous
import jax
import jax.numpy as jnp
from jax import lax
from jax.experimental import pallas as pl
from jax.experimental.pallas import tpu as pltpu
from jax.experimental.pallas import tpu_sc as plsc

_THRESH = 0.05
_TOP_N = 1000
_NT = 16
_SHARD = 1310720 // _NT
_NV = _SHARD // 16
_OUT_PAD = 1024
_MIN32 = -(2 ** 31)


def _sigmoid_mask_t(x):
    N, C, HW = x.shape

    def body(x_ref, o_ref):
        s = jax.nn.sigmoid(jnp.transpose(x_ref[0], (1, 0)))
        o_ref[0] = jnp.where(s > _THRESH, s, -1.0)

    blk = 2048
    return pl.pallas_call(
        body,
        grid=(N, HW // blk),
        in_specs=[pl.BlockSpec((1, C, blk), lambda i, j: (i, 0, j))],
        out_specs=pl.BlockSpec((1, blk, C), lambda i, j: (i, j, 0)),
        out_shape=jax.ShapeDtypeStruct((N, HW, C), x.dtype),
    )(x)


def _sc_body(masked_hbm, outS_hbm, outI_hbm,
             shard, hist, local, grid, bufK, bufI, othersK, othersI,
             pos_pub, rank_buf, score_buf, idx_buf, cntcopy,
             histgrid_sm, candK_sm, candI_sm, cntgrid_sm,
             outS_sm, outI_sm):
    c = lax.axis_index("c")
    s = lax.axis_index("s")
    lanes = lax.iota(jnp.int32, 16)
    lane256 = lanes * 257
    ones = jnp.ones((16,), jnp.int32)

    def splat(v):
        return jnp.full((16,), v, jnp.int32)

    def extract(vec, lane):
        return jnp.max(jnp.where(lanes == lane, vec, _MIN32))

    pltpu.sync_copy(masked_hbm.at[c, pl.ds(s * _SHARD, _SHARD)], shard)

    def clear_hist():
        def clr(w, _):
            for k in range(8):
                hist[pl.ds(w * 128 + k * 16, 16)] = jnp.zeros((16,), jnp.int32)
            return 0

        lax.fori_loop(0, 33, clr, 0)

    def reduce_and_decide(R):
        def red_w(w, _):
            acc = hist[pl.ds(w * 16, 16)]
            for lq in range(1, 16):
                acc = acc + hist[pl.ds(lq * 257 + w * 16, 16)]
            local[pl.ds(w * 16, 16)] = acc
            return 0

        lax.fori_loop(0, 16, red_w, 0)

        pltpu.sync_copy(local, histgrid_sm.at[pl.ds(s * 256, 256)])
        plsc.subcore_barrier()
        pltpu.sync_copy(histgrid_sm, grid)
        plsc.subcore_barrier()

        def tot_w(w, _):
            acc = grid[pl.ds(w * 16, 16)]
            for r in range(1, 16):
                acc = acc + grid[pl.ds(r * 256 + w * 16, 16)]
            local[pl.ds(w * 16, 16)] = acc
            return 0

        lax.fori_loop(0, 16, tot_w, 0)

        def walk(i, wc):
            R_w, b_w, done = wc
            blk = 15 - i
            v = local[pl.ds(blk * 16, 16)]
            ssum = jnp.sum(v)
            csum = plsc.cumsum(v)
            sfx = splat(ssum) - csum + v
            ge = sfx >= splat(R_w)
            m = jnp.max(plsc.all_reduce_population_count(ge)) - 1
            found = jnp.logical_and(jnp.logical_not(done), R_w <= ssum)
            vm = extract(v, m)
            sfxm = extract(sfx, m)
            R_in = R_w - (sfxm - vm)
            b_new = jnp.where(found, blk * 16 + m, b_w)
            R_new = jnp.where(found, R_in,
                              jnp.where(done, R_w, R_w - ssum))
            return (R_new, b_new, jnp.logical_or(done, found))

        R_f, b_l, _ = lax.fori_loop(
            0, 16, walk, (R, jnp.int32(0), jnp.bool_(False)))
        return R_f, b_l

    clear_hist()

    def scan0(i, _):
        for k in range(8):
            off = i * 128 + k * 16
            b = lax.bitcast_convert_type(shard[pl.ds(off, 16)], jnp.int32)
            key = jnp.where(b < 0,
                            jnp.bitwise_xor(jnp.bitwise_not(b), _MIN32), b)
            shard[pl.ds(off, 16)] = lax.bitcast_convert_type(key, jnp.float32)
            u = jnp.bitwise_xor(key, _MIN32)
            digit = jnp.bitwise_and(lax.shift_right_logical(u, 24), 255)
            plsc.addupdate_scatter(hist, [lane256 + digit], ones)
        return 0

    lax.fori_loop(0, _NV // 8, scan0, 0)
    R_0, b_0 = reduce_and_decide(jnp.int32(_TOP_N))
    pfx0 = lax.shift_left(b_0, 24)

    def level_body(l, carry):
        pfx_u, maskbits, shift, R, my_take = carry
        clear_hist()
        pfx_s = splat(pfx_u)
        mb_s = splat(maskbits)
        sh_s = splat(shift)

        def scan(i, _):
            ks = []
            matches = []
            anym = None
            for k in range(8):
                kk = lax.bitcast_convert_type(
                    shard[pl.ds(i * 128 + k * 16, 16)], jnp.int32)
                u = jnp.bitwise_xor(kk, _MIN32)
                m_ = jnp.bitwise_and(u, mb_s) == pfx_s
                ks.append(u)
                matches.append(m_)
                anym = m_ if anym is None else jnp.logical_or(anym, m_)

            @pl.when(jnp.any(anym))
            def _do():
                for k in range(8):
                    digit = jnp.bitwise_and(
                        lax.shift_right_logical(ks[k], sh_s), 255)
                    plsc.addupdate_scatter(hist, [lane256 + digit], ones,
                                           mask=matches[k])
            return 0

        lax.fori_loop(0, _NV // 8, scan, 0)
        R_f, b_l = reduce_and_decide(R)

        def take_fn(_):
            b3a = (b_l // 16) * 16
            b3m = b_l - b3a

            def ev_r(r, ev):
                row = grid[pl.ds(r * 256 + b3a, 16)]
                val = extract(row, b3m)
                return jnp.where(lanes == r, splat(val), ev)

            evec = lax.fori_loop(0, 16, ev_r, jnp.zeros((16,), jnp.int32))
            pecx = plsc.cumsum(evec) - evec
            tk = jnp.clip(splat(R_f) - pecx, 0, evec)
            return extract(tk, s)

        my_take_new = lax.cond(l == 3, take_fn, lambda _: my_take, 0)
        pfx_new = jnp.bitwise_or(pfx_u, lax.shift_left(b_l, shift))
        mb_new = jnp.bitwise_or(lax.shift_right_logical(maskbits, 8),
                                jnp.int32(-16777216))
        return (pfx_new, mb_new, shift - 8, R_f, my_take_new)

    pfx_u, _, _, R3, my_take = lax.fori_loop(
        1, 4, level_body,
        (pfx0, jnp.int32(-16777216), jnp.int32(16), R_0, jnp.int32(0)))
    t_key = jnp.bitwise_xor(pfx_u, _MIN32)

    t_s = splat(t_key)
    take_s = splat(my_take)
    base = s * _SHARD

    def fscan(i, carry):
        cnt_s, cntE_s = carry
        ks = []
        gts = []
        eqs = []
        anyv = None
        for k in range(8):
            kk = lax.bitcast_convert_type(
                shard[pl.ds(i * 128 + k * 16, 16)], jnp.int32)
            g = kk > t_s
            e = kk == t_s
            ks.append(kk)
            gts.append(g)
            eqs.append(e)
            h = jnp.logical_or(g, e)
            anyv = h if anyv is None else jnp.logical_or(anyv, h)

        def slow(cc):
            cnt_s, cntE_s = cc
            for k in range(8):
                gt, eq = gts[k], eqs[k]
                idxv = splat(base + i * 128 + k * 16) + lanes
                gti = gt.astype(jnp.int32)
                pg = plsc.cumsum(gti) - gti
                plsc.store_scatter(bufK, [cnt_s + pg], ks[k], mask=gt)
                plsc.store_scatter(bufI, [cnt_s + pg], idxv, mask=gt)
                ng = plsc.all_reduce_population_count(gt)
                eqi = eq.astype(jnp.int32)
                pe = plsc.cumsum(eqi) - eqi
                eff = jnp.logical_and(eq, (cntE_s + pe) < take_s)
                effi = eff.astype(jnp.int32)
                pf = plsc.cumsum(effi) - effi
                off2 = cnt_s + ng + pf
                plsc.store_scatter(bufK, [off2], ks[k], mask=eff)
                plsc.store_scatter(bufI, [off2], idxv, mask=eff)
                ne = plsc.all_reduce_population_count(eff)
                cnt_s = cnt_s + ng + ne
                cntE_s = cntE_s + ne
            return (cnt_s, cntE_s)

        return lax.cond(jnp.any(anyv), slow, lambda cc: cc, (cnt_s, cntE_s))

    cnt_spl, _ = lax.fori_loop(
        0, _NV // 8, fscan,
        (jnp.zeros((16,), jnp.int32), jnp.zeros((16,), jnp.int32)))
    cnt = jnp.max(cnt_spl)

    local[pl.ds(0, 16)] = splat(cnt)
    pltpu.sync_copy(local.at[pl.ds(0, 16)], cntgrid_sm.at[pl.ds(s * 16, 16)])
    plsc.subcore_barrier()
    pltpu.sync_copy(cntgrid_sm, cntcopy)

    def cv_r(r, cv):
        row = cntcopy[pl.ds(r * 16, 16)]
        return jnp.where(lanes == r, row, cv)

    cvec = lax.fori_loop(0, 16, cv_r, jnp.zeros((16,), jnp.int32))
    offs = plsc.cumsum(cvec) - cvec
    my_off = extract(offs, s)

    for j in range(8):
        for w in range(8):
            kk = j * 128 + w * 16
            kvec = splat(kk) + lanes
            in_real = kvec < splat(cnt)
            pos = jnp.where(in_real, splat(my_off) + kvec,
                            splat(_OUT_PAD) + kvec)
            pos_pub[j, pl.ds(w * 16, 16)] = pos
    for j in range(8):
        pltpu.sync_copy(bufK.at[pl.ds(j * 128, 128)], candK_sm.at[pos_pub.at[j]])
        pltpu.sync_copy(bufI.at[pl.ds(j * 128, 128)], candI_sm.at[pos_pub.at[j]])
    plsc.subcore_barrier()
    pltpu.sync_copy(candK_sm.at[pl.ds(0, _OUT_PAD)], othersK)
    pltpu.sync_copy(candI_sm.at[pl.ds(0, _OUT_PAD)], othersI)

    my_lo = s * 64

    def rank_j(j, _):
        p = my_lo + j
        pa = (p // 16) * 16
        pm = p - pa
        kj = extract(othersK[pl.ds(pa, 16)], pm)
        ij = extract(othersI[pl.ds(pa, 16)], pm)
        kj_s = splat(kj)
        ij_s = splat(ij)

        def sweep(w, acc):
            ok = othersK[pl.ds(w * 16, 16)]
            oi = othersI[pl.ds(w * 16, 16)]
            valid = (splat(w * 16) + lanes) < _TOP_N
            beat = jnp.logical_or(
                ok > kj_s, jnp.logical_and(ok == kj_s, oi < ij_s))
            return acc + plsc.all_reduce_population_count(
                jnp.logical_and(beat, valid))

        racc = lax.fori_loop(0, 63, sweep, jnp.zeros((16,), jnp.int32))
        rank = jnp.max(racc)
        rank = jnp.where(p < _TOP_N, rank, p)
        plsc.store_scatter(rank_buf, [splat(j)], splat(rank),
                           mask=lanes == 0)
        return 0

    lax.fori_loop(0, 64, rank_j, 0)

    for m in range(4):
        kv = lax.bitcast_convert_type(othersK[pl.ds(my_lo + m * 16, 16)], jnp.int32)
        bits = jnp.where(kv < 0,
                         jnp.bitwise_not(jnp.bitwise_xor(kv, _MIN32)), kv)
        score_buf[pl.ds(m * 16, 16)] = lax.bitcast_convert_type(bits, jnp.float32)
        idx_buf[pl.ds(m * 16, 16)] = othersI[pl.ds(my_lo + m * 16, 16)]
    pltpu.sync_copy(score_buf, outS_sm.at[rank_buf])
    pltpu.sync_copy(idx_buf, outI_sm.at[rank_buf])
    plsc.subcore_barrier()

    @pl.when(s == 0)
    def _write_out():
        pltpu.sync_copy(outS_sm, outS_hbm.at[c])
        pltpu.sync_copy(outI_sm, outI_hbm.at[c])


def _sc_topk(masked):
    mesh = plsc.VectorSubcoreMesh(core_axis_name="c", subcore_axis_name="s")
    f32 = jnp.float32
    i32 = jnp.int32
    return pl.kernel(
        _sc_body,
        out_type=[jax.ShapeDtypeStruct((2, _OUT_PAD), f32),
                  jax.ShapeDtypeStruct((2, _OUT_PAD), i32)],
        mesh=mesh,
        compiler_params=pltpu.CompilerParams(needs_layout_passes=False),
        scratch_types=[
            pltpu.VMEM((_SHARD,), f32),
            pltpu.VMEM((4224,), i32),
            pltpu.VMEM((256,), i32),
            pltpu.VMEM((4096,), i32),
            pltpu.VMEM((1024,), i32),
            pltpu.VMEM((1024,), i32),
            pltpu.VMEM((1024,), i32),
            pltpu.VMEM((1024,), i32),
            pltpu.VMEM((8, 128), i32),
            pltpu.VMEM((64,), i32),
            pltpu.VMEM((64,), f32),
            pltpu.VMEM((64,), i32),
            pltpu.VMEM((256,), i32),
            pltpu.VMEM_SHARED((4096,), i32),
            pltpu.VMEM_SHARED((2 * _OUT_PAD,), i32),
            pltpu.VMEM_SHARED((2 * _OUT_PAD,), i32),
            pltpu.VMEM_SHARED((256,), i32),
            pltpu.VMEM_SHARED((_OUT_PAD,), f32),
            pltpu.VMEM_SHARED((_OUT_PAD,), i32),
        ],
    )(masked)


def kernel(locations, box_cls_set, box_regression, centerness, image_sizes):
    N, C, H, W = box_cls_set.shape
    HW = H * W
    masked = _sigmoid_mask_t(
        box_cls_set.reshape(N, C, HW)).reshape(N, HW * C)

    outS, outI = _sc_topk(masked)
    top_scores = outS[:, :_TOP_N]
    top_idx = outI[:, :_TOP_N]

    box_reg = jnp.transpose(box_regression, (0, 2, 3, 1)).reshape(N, HW, 4)
    loc_idx = top_idx // C
    labels = top_idx % C + 1
    valid = top_scores > _THRESH
    per_loc = locations[loc_idx]
    per_reg = jnp.take_along_axis(box_reg, loc_idx[..., None], axis=1)
    x1 = per_loc[..., 0] - per_reg[..., 0]
    y1 = per_loc[..., 1] - per_reg[..., 1]
    x2 = per_loc[..., 0] + per_reg[..., 2]
    y2 = per_loc[..., 1] + per_reg[..., 3]
    h_img = image_sizes[:, 0].astype(jnp.float32)[:, None]
    w_img = image_sizes[:, 1].astype(jnp.float32)[:, None]
    x1 = jnp.clip(x1, 0.0, w_img - 1.0)
    x2 = jnp.clip(x2, 0.0, w_img - 1.0)
    y1 = jnp.clip(y1, 0.0, h_img - 1.0)
    y2 = jnp.clip(y2, 0.0, h_img - 1.0)
    detections = jnp.stack([x1, y1, x2, y2], axis=-1)
    ws = x2 - x1 + 1.0
    hs = y2 - y1 + 1.0
    keep = (ws >= 0) & (hs >= 0)
    scores = jnp.where(valid & keep, top_scores, 0.0)
    return detections, scores, labels, per_loc
